# Initial kernel scaffold; baseline (speedup 1.0000x reference)
import jax
import jax.numpy as jnp
from jax.experimental import pallas as pl


def kernel(x, kohonen_weights, grossberg_weights):
    raise NotImplementedError("write your pallas kernel here")


# speedup vs baseline: 2.7189x; 2.7189x over previous
import jax
import jax.numpy as jnp
from jax.experimental import pallas as pl
from jax.experimental.pallas import tpu as pltpu


def _cpn_body(x_ref, xsq_ref, wsq_ref, kwt_ref, gwt_ref, out_ref, win_ref):
    cross = jnp.dot(x_ref[...], kwt_ref[...], preferred_element_type=jnp.float32)
    dist = jnp.sqrt(jnp.maximum(xsq_ref[...] + wsq_ref[...] - 2.0 * cross, 0.0))
    bt, h = cross.shape
    dmin = jnp.min(dist, axis=1, keepdims=True)
    iota = jax.lax.broadcasted_iota(jnp.int32, (bt, h), 1)
    win = jnp.min(jnp.where(dist == dmin, iota, h), axis=1).astype(jnp.int32)
    win_ref[...] = win[:, None]
    one_hot = (iota == win[:, None])
    out_ref[...] = jnp.dot(one_hot.astype(jnp.float32), gwt_ref[...],
                           preferred_element_type=jnp.float32)


def kernel(x, kohonen_weights, grossberg_weights):
    batch, in_dim = x.shape
    hidden = kohonen_weights.shape[0]
    out_dim = grossberg_weights.shape[0]
    x_sq = jnp.sum(x * x, axis=1, keepdims=True)
    w_sq = jnp.sum(kohonen_weights * kohonen_weights, axis=1)[None, :]
    kwt = kohonen_weights.T
    gwt = grossberg_weights.T

    bt = 1024
    grid = (batch // bt,)
    out, win = pl.pallas_call(
        _cpn_body,
        grid=grid,
        in_specs=[
            pl.BlockSpec((bt, in_dim), lambda i: (i, 0)),
            pl.BlockSpec((bt, 1), lambda i: (i, 0)),
            pl.BlockSpec((1, hidden), lambda i: (0, 0)),
            pl.BlockSpec((in_dim, hidden), lambda i: (0, 0)),
            pl.BlockSpec((hidden, out_dim), lambda i: (0, 0)),
        ],
        out_specs=[
            pl.BlockSpec((bt, out_dim), lambda i: (i, 0)),
            pl.BlockSpec((bt, 1), lambda i: (i, 0)),
        ],
        out_shape=[
            jax.ShapeDtypeStruct((batch, out_dim), jnp.float32),
            jax.ShapeDtypeStruct((batch, 1), jnp.int32),
        ],
    )(x, x_sq, w_sq, kwt, gwt)
    return (out, win[:, 0])
